# baseline (speedup 1.0000x reference)
import functools

import jax
import jax.numpy as jnp
from jax import lax
from jax.experimental import pallas as pl
from jax.experimental.pallas import tpu as pltpu
from jax.experimental.pallas import tpu_sc as plsc

N_NODES = 10000
D = 128
N_EDGES = 320000
NC = 2
NS = 16
NW = NC * NS
B = 64
K = 4
NB = 160
EPT = NB * B
E_PAD = NW * EPT
N_PAD = 10240
ROWS_PER_TILE = N_PAD // NS


def _mm_body(x_ref, wt_ref, b_ref, o_ref):
    o_ref[...] = (
        jnp.dot(x_ref[...], wt_ref[...], preferred_element_type=jnp.float32)
        + b_ref[...]
    )


def _support(x, wt, b2):
    return pl.pallas_call(
        _mm_body,
        grid=(10,),
        in_specs=[
            pl.BlockSpec((N_NODES // 10, D), lambda i: (i, 0)),
            pl.BlockSpec((D, D), lambda i: (0, 0)),
            pl.BlockSpec((1, D), lambda i: (0, 0)),
        ],
        out_specs=pl.BlockSpec((N_NODES // 10, D), lambda i: (i, 0)),
        out_shape=jax.ShapeDtypeStruct((N_NODES, D), jnp.float32),
    )(x, wt, b2)


def _add_body(p_ref, o_ref):
    o_ref[...] = p_ref[0] + p_ref[1]


def _combine(p):
    return pl.pallas_call(
        _add_body,
        grid=(5,),
        in_specs=[pl.BlockSpec((2, N_PAD // 5, D), lambda i: (0, i, 0))],
        out_specs=pl.BlockSpec((N_PAD // 5, D), lambda i: (i, 0)),
        out_shape=jax.ShapeDtypeStruct((N_PAD, D), jnp.float32),
    )(p)


_MESH = plsc.VectorSubcoreMesh(core_axis_name="c", subcore_axis_name="s")


@functools.partial(
    pl.kernel,
    out_type=jax.ShapeDtypeStruct((NC, N_PAD, D), jnp.float32),
    mesh=_MESH,
    scratch_types=[
        pltpu.VMEM((K, 3, B), jnp.int32),
        pltpu.VMEM((K, B, D), jnp.float32),
        pltpu.VMEM_SHARED((N_PAD, D), jnp.float32),
        pltpu.SemaphoreType.DMA,
        pltpu.SemaphoreType.DMA,
        pltpu.SemaphoreType.DMA,
        pltpu.SemaphoreType.DMA,
        pltpu.SemaphoreType.DMA,
        pltpu.SemaphoreType.DMA,
    ],
)
def _spmm(support_hbm, idx_hbm, out_hbm, idxb, rows, acc,
          sem_idx, sem_g0, sem_g1, sem_g2, sem_g3, sem_sc):
    c = lax.axis_index("c")
    s = lax.axis_index("s")
    wid = s * NC + c
    sem_g = [sem_g0, sem_g1, sem_g2, sem_g3]

    zeros16 = jnp.zeros((16,), jnp.float32)

    def _zrow(r, carry):
        for j in range(D // 16):
            rows[0, r, pl.ds(j * 16, 16)] = zeros16
        return carry

    lax.fori_loop(0, B, _zrow, 0)
    for k in range(ROWS_PER_TILE // B):
        pltpu.sync_copy(
            rows.at[0], acc.at[pl.ds(s * ROWS_PER_TILE + k * B, B)])
    plsc.subcore_barrier()

    for h in range(K - 1):
        pltpu.sync_copy(idx_hbm.at[wid, h], idxb.at[h])
        pltpu.async_copy(support_hbm.at[idxb.at[h, 0]], rows.at[h], sem_g[h])

    def _outer(t, carry):
        for k in range(K):
            g = t * K + k
            r = (k + K - 1) % K
            pltpu.make_async_copy(
                support_hbm.at[idxb.at[k, 0]], rows.at[k], sem_g[k]).wait()

            for e in range(B):
                if e % 16 == 0:
                    w16 = lax.bitcast_convert_type(
                        idxb[k, 2, pl.ds(e, 16)], jnp.float32)
                wspl = lax.gather(
                    w16, jnp.full((16, 1), e % 16, jnp.int32),
                    lax.GatherDimensionNumbers(
                        offset_dims=(), collapsed_slice_dims=(0,),
                        start_index_map=(0,)),
                    slice_sizes=(1,),
                    mode=lax.GatherScatterMode.PROMISE_IN_BOUNDS)
                for j in range(D // 16):
                    sl = pl.ds(j * 16, 16)
                    rows[k, e, sl] = rows[k, e, sl] * wspl

            @pl.when(g > 0)
            def _():
                pltpu.make_async_copy(
                    rows.at[r], acc.at[idxb.at[r, 1]], sem_sc).wait()

            pltpu.async_copy(rows.at[k], acc.at[idxb.at[k, 1]], sem_sc,
                             add=True)

            @pl.when(g < NB - (K - 1))
            def _():
                pltpu.async_copy(idx_hbm.at[wid, g + K - 1], idxb.at[r],
                                 sem_idx)
                pltpu.make_async_copy(
                    idx_hbm.at[wid, g + K - 1], idxb.at[r], sem_idx).wait()
                pltpu.async_copy(
                    support_hbm.at[idxb.at[r, 0]], rows.at[r], sem_g[r])

        return carry

    lax.fori_loop(0, NB // K, _outer, 0)
    lastk = (NB - 1) % K
    pltpu.make_async_copy(
        rows.at[lastk], acc.at[idxb.at[lastk, 1]], sem_sc).wait()
    plsc.subcore_barrier()

    pltpu.sync_copy(
        acc.at[pl.ds(s * ROWS_PER_TILE, ROWS_PER_TILE)],
        out_hbm.at[c, pl.ds(s * ROWS_PER_TILE, ROWS_PER_TILE)])


def kernel(x, edge_index, edge_weight, W, b):
    support = _support(x, W.T, b.reshape(1, D))
    pad = E_PAD - N_EDGES
    srci = jnp.pad(edge_index[1].astype(jnp.int32), (0, pad))
    dsti = jnp.pad(edge_index[0].astype(jnp.int32), (0, pad))
    wi = lax.bitcast_convert_type(
        jnp.pad(edge_weight.astype(jnp.float32), (0, pad)), jnp.int32)
    packed = jnp.stack(
        [srci.reshape(NW, NB, B), dsti.reshape(NW, NB, B),
         wi.reshape(NW, NB, B)], axis=2)
    partials = _spmm(support, packed)
    return _combine(partials)[:N_NODES]


# speedup vs baseline: 1.6754x; 1.6754x over previous
import functools

import jax
import jax.numpy as jnp
from jax import lax
from jax.experimental import pallas as pl
from jax.experimental.pallas import tpu as pltpu
from jax.experimental.pallas import tpu_sc as plsc

N_NODES = 10000
D = 128
N_EDGES = 320000
NC = 2
NS = 16
NW = NC * NS
B = 128
NB0 = 111
NB1 = 46
EPT0 = NB0 * B
EPT1 = NB1 * B
E_PAD = NS * (EPT0 + EPT1)
N_PAD = 10240
ROWS_PER_TILE = N_PAD // NS


def _mm_body(x_ref, wt_ref, b_ref, o_ref):
    o_ref[...] = (
        jnp.dot(x_ref[...], wt_ref[...], preferred_element_type=jnp.float32)
        + b_ref[...]
    )


def _support(x, wt, b2):
    return pl.pallas_call(
        _mm_body,
        grid=(10,),
        in_specs=[
            pl.BlockSpec((N_NODES // 10, D), lambda i: (i, 0)),
            pl.BlockSpec((D, D), lambda i: (0, 0)),
            pl.BlockSpec((1, D), lambda i: (0, 0)),
        ],
        out_specs=pl.BlockSpec((N_NODES // 10, D), lambda i: (i, 0)),
        out_shape=jax.ShapeDtypeStruct((N_NODES, D), jnp.float32),
    )(x, wt, b2)


def _add_body(p_ref, o_ref):
    o_ref[...] = p_ref[0] + p_ref[1]


def _combine(p):
    return pl.pallas_call(
        _add_body,
        grid=(5,),
        in_specs=[pl.BlockSpec((2, N_PAD // 5, D), lambda i: (0, i, 0))],
        out_specs=pl.BlockSpec((N_PAD // 5, D), lambda i: (i, 0)),
        out_shape=jax.ShapeDtypeStruct((N_PAD, D), jnp.float32),
    )(p)


_MESH = plsc.VectorSubcoreMesh(core_axis_name="c", subcore_axis_name="s")


@functools.partial(
    pl.kernel,
    out_type=jax.ShapeDtypeStruct((NC, N_PAD, D), jnp.float32),
    mesh=_MESH,
    scratch_types=[
        pltpu.VMEM((2, 3, B), jnp.int32),
        pltpu.VMEM((2, B, D), jnp.float32),
        pltpu.VMEM_SHARED((N_PAD, D), jnp.float32),
        pltpu.SemaphoreType.DMA,
        pltpu.SemaphoreType.DMA,
        pltpu.SemaphoreType.DMA,
    ],
)
def _spmm(support_hbm, idx_hbm, out_hbm, idxb, rows, acc,
          sem_idx, sem_g, sem_sc):
    c = lax.axis_index("c")
    s = lax.axis_index("s")
    wid = s * NC + c
    nb = jnp.where(c == 0, NB0, NB1)

    zeros16 = jnp.zeros((16,), jnp.float32)

    def _zrow(r, carry):
        for j in range(D // 16):
            rows[0, r, pl.ds(j * 16, 16)] = zeros16
        return carry

    lax.fori_loop(0, B, _zrow, 0)
    for k in range(ROWS_PER_TILE // B):
        pltpu.sync_copy(
            rows.at[0], acc.at[pl.ds(s * ROWS_PER_TILE + k * B, B)])
    plsc.subcore_barrier()

    pltpu.sync_copy(idx_hbm.at[wid, 0], idxb.at[0])
    pltpu.async_copy(support_hbm.at[idxb.at[0, 0]], rows.at[0], sem_g)

    def _body(g, carry):
        p = lax.rem(g, 2)
        q = 1 - p
        pltpu.make_async_copy(
            support_hbm.at[idxb.at[p, 0]], rows.at[p], sem_g).wait()

        @pl.when(g > 0)
        def _():
            pltpu.make_async_copy(
                rows.at[q], acc.at[idxb.at[q, 1]], sem_sc).wait()

        @pl.when(g < nb - 1)
        def _():
            pltpu.async_copy(idx_hbm.at[wid, g + 1], idxb.at[q], sem_idx)

        for e in range(B):
            if e % 16 == 0:
                w16 = lax.bitcast_convert_type(
                    idxb[p, 2, pl.ds(e, 16)], jnp.float32)
            wspl = lax.gather(
                w16, jnp.full((16, 1), e % 16, jnp.int32),
                lax.GatherDimensionNumbers(
                    offset_dims=(), collapsed_slice_dims=(0,),
                    start_index_map=(0,)),
                slice_sizes=(1,),
                mode=lax.GatherScatterMode.PROMISE_IN_BOUNDS)
            for j in range(D // 16):
                sl = pl.ds(j * 16, 16)
                rows[p, e, sl] = rows[p, e, sl] * wspl

        pltpu.async_copy(rows.at[p], acc.at[idxb.at[p, 1]], sem_sc, add=True)

        @pl.when(g < nb - 1)
        def _():
            pltpu.make_async_copy(
                idx_hbm.at[wid, g + 1], idxb.at[q], sem_idx).wait()
            pltpu.async_copy(
                support_hbm.at[idxb.at[q, 0]], rows.at[q], sem_g)

        return carry

    lax.fori_loop(0, nb, _body, 0)
    lastp = lax.rem(nb - 1, 2)
    pltpu.make_async_copy(
        rows.at[lastp], acc.at[idxb.at[lastp, 1]], sem_sc).wait()
    plsc.subcore_barrier()

    pltpu.sync_copy(
        acc.at[pl.ds(s * ROWS_PER_TILE, ROWS_PER_TILE)],
        out_hbm.at[c, pl.ds(s * ROWS_PER_TILE, ROWS_PER_TILE)])


def _layout(v):
    a0 = v[:NS * EPT0].reshape(NS, NB0, B)
    a1 = v[NS * EPT0:].reshape(NS, NB1, B)
    a1 = jnp.pad(a1, ((0, 0), (0, NB0 - NB1), (0, 0)))
    return jnp.stack([a0, a1], axis=1).reshape(NW, NB0, B)


def kernel(x, edge_index, edge_weight, W, b):
    support = _support(x, W.T, b.reshape(1, D))
    pad = E_PAD - N_EDGES
    srci = jnp.pad(edge_index[1].astype(jnp.int32), (0, pad))
    dsti = jnp.pad(edge_index[0].astype(jnp.int32), (0, pad))
    wi = lax.bitcast_convert_type(
        jnp.pad(edge_weight.astype(jnp.float32), (0, pad)), jnp.int32)
    packed = jnp.stack(
        [_layout(srci), _layout(dsti), _layout(wi)], axis=2)
    partials = _spmm(support, packed)
    return _combine(partials)[:N_NODES]


# speedup vs baseline: 1.8123x; 1.0817x over previous
import functools

import jax
import jax.numpy as jnp
from jax import lax
from jax.experimental import pallas as pl
from jax.experimental.pallas import tpu as pltpu
from jax.experimental.pallas import tpu_sc as plsc

N_NODES = 10000
D = 128
N_EDGES = 320000
NC = 2
NS = 16
NW = NC * NS
B = 128
NB0 = 98
NB1 = 59
EPT0 = NB0 * B
EPT1 = NB1 * B
E_PAD = NS * (EPT0 + EPT1)
N_PAD = 10240
ROWS_PER_TILE = N_PAD // NS


def _mm_body(x_ref, wt_ref, b_ref, o_ref):
    o_ref[...] = (
        jnp.dot(x_ref[...], wt_ref[...], preferred_element_type=jnp.float32)
        + b_ref[...]
    )


def _support(x, wt, b2):
    return pl.pallas_call(
        _mm_body,
        grid=(10,),
        in_specs=[
            pl.BlockSpec((N_NODES // 10, D), lambda i: (i, 0)),
            pl.BlockSpec((D, D), lambda i: (0, 0)),
            pl.BlockSpec((1, D), lambda i: (0, 0)),
        ],
        out_specs=pl.BlockSpec((N_NODES // 10, D), lambda i: (i, 0)),
        out_shape=jax.ShapeDtypeStruct((N_NODES, D), jnp.float32),
    )(x, wt, b2)


def _add_body(p_ref, o_ref):
    o_ref[...] = p_ref[0] + p_ref[1]


def _combine(p):
    return pl.pallas_call(
        _add_body,
        grid=(5,),
        in_specs=[pl.BlockSpec((2, N_PAD // 5, D), lambda i: (0, i, 0))],
        out_specs=pl.BlockSpec((N_PAD // 5, D), lambda i: (i, 0)),
        out_shape=jax.ShapeDtypeStruct((N_PAD, D), jnp.float32),
    )(p)


_MESH = plsc.VectorSubcoreMesh(core_axis_name="c", subcore_axis_name="s")


@functools.partial(
    pl.kernel,
    out_type=jax.ShapeDtypeStruct((NC, N_PAD, D), jnp.float32),
    mesh=_MESH,
    scratch_types=[
        pltpu.VMEM((2, 3, B), jnp.int32),
        pltpu.VMEM((2, B, D), jnp.float32),
        pltpu.VMEM_SHARED((N_PAD, D), jnp.float32),
        pltpu.SemaphoreType.DMA,
        pltpu.SemaphoreType.DMA,
        pltpu.SemaphoreType.DMA,
    ],
)
def _spmm(support_hbm, idx_hbm, out_hbm, idxb, rows, acc,
          sem_idx, sem_g, sem_sc):
    c = lax.axis_index("c")
    s = lax.axis_index("s")
    wid = s * NC + c
    nb = jnp.where(c == 0, NB0, NB1)

    zeros16 = jnp.zeros((16,), jnp.float32)

    def _zrow(r, carry):
        for j in range(D // 16):
            rows[0, r, pl.ds(j * 16, 16)] = zeros16
        return carry

    lax.fori_loop(0, B, _zrow, 0)
    for k in range(ROWS_PER_TILE // B):
        pltpu.sync_copy(
            rows.at[0], acc.at[pl.ds(s * ROWS_PER_TILE + k * B, B)])
    plsc.subcore_barrier()

    pltpu.sync_copy(idx_hbm.at[wid, 0], idxb.at[0])
    pltpu.async_copy(support_hbm.at[idxb.at[0, 0]], rows.at[0], sem_g)

    def _body(g, carry):
        p = lax.rem(g, 2)
        q = 1 - p
        pltpu.make_async_copy(
            support_hbm.at[idxb.at[p, 0]], rows.at[p], sem_g).wait()

        @pl.when(g > 0)
        def _():
            pltpu.make_async_copy(
                rows.at[q], acc.at[idxb.at[q, 1]], sem_sc).wait()

        @pl.when(g < nb - 1)
        def _():
            pltpu.async_copy(idx_hbm.at[wid, g + 1], idxb.at[q], sem_idx)

        for e in range(B):
            if e % 16 == 0:
                w16 = lax.bitcast_convert_type(
                    idxb[p, 2, pl.ds(e, 16)], jnp.float32)
            wspl = lax.gather(
                w16, jnp.full((16, 1), e % 16, jnp.int32),
                lax.GatherDimensionNumbers(
                    offset_dims=(), collapsed_slice_dims=(0,),
                    start_index_map=(0,)),
                slice_sizes=(1,),
                mode=lax.GatherScatterMode.PROMISE_IN_BOUNDS)
            for j in range(D // 16):
                sl = pl.ds(j * 16, 16)
                rows[p, e, sl] = rows[p, e, sl] * wspl

        pltpu.async_copy(rows.at[p], acc.at[idxb.at[p, 1]], sem_sc, add=True)

        @pl.when(g < nb - 1)
        def _():
            pltpu.make_async_copy(
                idx_hbm.at[wid, g + 1], idxb.at[q], sem_idx).wait()
            pltpu.async_copy(
                support_hbm.at[idxb.at[q, 0]], rows.at[q], sem_g)

        return carry

    lax.fori_loop(0, nb, _body, 0)
    lastp = lax.rem(nb - 1, 2)
    pltpu.make_async_copy(
        rows.at[lastp], acc.at[idxb.at[lastp, 1]], sem_sc).wait()
    plsc.subcore_barrier()

    pltpu.sync_copy(
        acc.at[pl.ds(s * ROWS_PER_TILE, ROWS_PER_TILE)],
        out_hbm.at[c, pl.ds(s * ROWS_PER_TILE, ROWS_PER_TILE)])


def _layout(v):
    a0 = v[:NS * EPT0].reshape(NS, NB0, B)
    a1 = v[NS * EPT0:].reshape(NS, NB1, B)
    a1 = jnp.pad(a1, ((0, 0), (0, NB0 - NB1), (0, 0)))
    return jnp.stack([a0, a1], axis=1).reshape(NW, NB0, B)


def kernel(x, edge_index, edge_weight, W, b):
    support = _support(x, W.T, b.reshape(1, D))
    pad = E_PAD - N_EDGES
    srci = jnp.pad(edge_index[1].astype(jnp.int32), (0, pad))
    dsti = jnp.pad(edge_index[0].astype(jnp.int32), (0, pad))
    wi = lax.bitcast_convert_type(
        jnp.pad(edge_weight.astype(jnp.float32), (0, pad)), jnp.int32)
    packed = jnp.stack(
        [_layout(srci), _layout(dsti), _layout(wi)], axis=2)
    partials = _spmm(support, packed)
    return _combine(partials)[:N_NODES]
